# Initial kernel scaffold; baseline (speedup 1.0000x reference)
import jax
import jax.numpy as jnp
from jax.experimental import pallas as pl


def kernel(user_ids, item_ids, content_features, edge_index, user_table, item_table, Wc, bc, W1s, W1d, a1s, a1d, bconv1, W2s, W2d, a2s, a2d, bconv2, Wp1, bp1, Wp2, bp2):
    raise NotImplementedError("write your pallas kernel here")


# speedup vs baseline: 59.1250x; 59.1250x over previous
import functools

import jax
import jax.numpy as jnp
from jax import lax
from jax.experimental import pallas as pl
from jax.experimental.pallas import tpu as pltpu
from jax.experimental.pallas import tpu_sc as plsc

NUM_USERS = 25000
NUM_ITEMS = 25000
N_NODES = NUM_USERS + NUM_ITEMS
EMB = 32
CF = 128
B = 16384
E = 800000

N_PAD = 51200
NPT = N_PAD // 16
NBLK_E = E // 128

_SC_MESH = plsc.VectorSubcoreMesh(
    core_axis_name="c", subcore_axis_name="s", num_cores=2, num_subcores=16
)
_SC_PARAMS = pltpu.CompilerParams(
    use_tc_tiling_on_sc=False, needs_layout_passes=False)

_IOTA16 = lambda: lax.iota(jnp.int32, 16)


def _splat(v):
    return jnp.full((16,), v, jnp.int32)


def _cemb_body(cf_ref, wc_ref, bc_ref, out_ref):
    out_ref[...] = (
        lax.dot_general(
            cf_ref[...], wc_ref[...], (((1,), (1,)), ((), ())),
            preferred_element_type=jnp.float32,
        )
        + bc_ref[...][None, :]
    )


def _cemb(cf, Wc, bc):
    return pl.pallas_call(
        _cemb_body,
        grid=(16,),
        in_specs=[
            pl.BlockSpec((1024, CF), lambda i: (i, 0)),
            pl.BlockSpec((EMB, CF), lambda i: (0, 0)),
            pl.BlockSpec((EMB,), lambda i: (0,)),
        ],
        out_specs=pl.BlockSpec((1024, EMB), lambda i: (i, 0)),
        out_shape=jax.ShapeDtypeStruct((B, EMB), jnp.float32),
    )(cf, Wc, bc)


def _l1pre_body(x_ref, ws_ref, wd_ref, as_ref, ad_ref, xs_out, asd_out):
    h = pl.program_id(0)
    xb = x_ref[...]
    xs = lax.dot_general(xb, ws_ref[...], (((1,), (1,)), ((), ())),
                         preferred_element_type=jnp.float32)
    xd = lax.dot_general(xb, wd_ref[...], (((1,), (1,)), ((), ())),
                         preferred_element_type=jnp.float32)
    am = as_ref[...]
    bm = ad_ref[...]
    av = jnp.where(h == 0, am[0], am[1])[:, None]
    bv = jnp.where(h == 0, bm[0], bm[1])[:, None]
    a_s = lax.dot_general(xs, av, (((1,), (0,)), ((), ())),
                          preferred_element_type=jnp.float32)
    a_d = lax.dot_general(xd, bv, (((1,), (0,)), ((), ())),
                          preferred_element_type=jnp.float32)
    xs_out[...] = xs
    asd_out[...] = jnp.concatenate(
        [a_s, a_d, jnp.zeros((a_s.shape[0], 6), jnp.float32)], axis=1)


def _l1pre(x_pad, W1s, W1d, a1s, a1d):
    return pl.pallas_call(
        _l1pre_body,
        grid=(2, 25),
        in_specs=[
            pl.BlockSpec((2048, 2 * EMB), lambda h, i: (i, 0)),
            pl.BlockSpec((EMB, 2 * EMB), lambda h, i: (h, 0)),
            pl.BlockSpec((EMB, 2 * EMB), lambda h, i: (h, 0)),
            pl.BlockSpec((2, EMB), lambda h, i: (0, 0)),
            pl.BlockSpec((2, EMB), lambda h, i: (0, 0)),
        ],
        out_specs=[
            pl.BlockSpec((2048, EMB), lambda h, i: (h * 25 + i, 0)),
            pl.BlockSpec((2048, 8), lambda h, i: (h * 25 + i, 0)),
        ],
        out_shape=[
            jax.ShapeDtypeStruct((2 * N_PAD, EMB), jnp.float32),
            jax.ShapeDtypeStruct((2 * N_PAD, 8), jnp.float32),
        ],
    )(x_pad, W1s, W1d, a1s, a1d)


def _l2pre_body(ha_ref, hb_ref, ws_ref, wd_ref, as_ref, ad_ref, xs_out, asd_out):
    xb = jnp.concatenate([ha_ref[...], hb_ref[...]], axis=1)
    xs = lax.dot_general(xb, ws_ref[...], (((1,), (1,)), ((), ())),
                         preferred_element_type=jnp.float32)
    xd = lax.dot_general(xb, wd_ref[...], (((1,), (1,)), ((), ())),
                         preferred_element_type=jnp.float32)
    av = as_ref[...][:, None]
    bv = ad_ref[...][:, None]
    a_s = lax.dot_general(xs, av, (((1,), (0,)), ((), ())),
                          preferred_element_type=jnp.float32)
    a_d = lax.dot_general(xd, bv, (((1,), (0,)), ((), ())),
                          preferred_element_type=jnp.float32)
    xs_out[...] = xs
    asd_out[...] = jnp.concatenate(
        [a_s, a_d, jnp.zeros((a_s.shape[0], 6), jnp.float32)], axis=1)


def _l2pre(h1A, W2s, W2d, a2s, a2d):
    return pl.pallas_call(
        _l2pre_body,
        grid=(25,),
        in_specs=[
            pl.BlockSpec((2048, EMB), lambda i: (i, 0)),
            pl.BlockSpec((2048, EMB), lambda i: (25 + i, 0)),
            pl.BlockSpec((EMB, 2 * EMB), lambda i: (0, 0)),
            pl.BlockSpec((EMB, 2 * EMB), lambda i: (0, 0)),
            pl.BlockSpec((EMB,), lambda i: (0,)),
            pl.BlockSpec((EMB,), lambda i: (0,)),
        ],
        out_specs=[
            pl.BlockSpec((2048, EMB), lambda i: (i, 0)),
            pl.BlockSpec((2048, 8), lambda i: (i, 0)),
        ],
        out_shape=[
            jax.ShapeDtypeStruct((N_PAD, EMB), jnp.float32),
            jax.ShapeDtypeStruct((N_PAD, 8), jnp.float32),
        ],
    )(h1A, h1A, W2s, W2d, a2s, a2d)


def _mlp_body(cb_ref, w1_ref, b1_ref, w2_ref, b2_ref, out_ref):
    hid = lax.dot_general(cb_ref[...], w1_ref[...], (((1,), (1,)), ((), ())),
                          preferred_element_type=jnp.float32)
    hid = jnp.maximum(hid + b1_ref[...][None, :], 0.0)
    out = lax.dot_general(hid, w2_ref[...], (((1,), (1,)), ((), ())),
                          preferred_element_type=jnp.float32)
    out_ref[...] = out + b2_ref[0]


def _mlp(comb, Wp1, bp1, Wp2, bp2):
    return pl.pallas_call(
        _mlp_body,
        grid=(16,),
        in_specs=[
            pl.BlockSpec((1024, 3 * EMB), lambda i: (i, 0)),
            pl.BlockSpec((EMB, 3 * EMB), lambda i: (0, 0)),
            pl.BlockSpec((EMB,), lambda i: (0,)),
            pl.BlockSpec((8, EMB), lambda i: (0, 0)),
            pl.BlockSpec((1,), lambda i: (0,)),
        ],
        out_specs=pl.BlockSpec((1024, 8), lambda i: (i, 0)),
        out_shape=jax.ShapeDtypeStruct((B, 8), jnp.float32),
    )(comb, Wp1, bp1, Wp2, bp2)


def _scale_chunk(bufA, colA, bufB, colB, rows_ref, den_store, scale):
    for g in range(8):
        idx16 = _IOTA16() + (16 * g)
        a_v = plsc.load_gather(bufA, [idx16, colA])
        b_v = plsc.load_gather(bufB, [idx16, colB])
        z = a_v + b_v
        z = jnp.where(z > 0, z, z * 0.2)
        ex = jnp.exp(z) * scale
        den_store(g, idx16, ex)
        for j in range(16):
            e = 16 * g + j
            exj = ex[j]
            v0 = rows_ref[e, pl.ds(0, 16)]
            rows_ref[e, pl.ds(0, 16)] = v0 * exj
            v1 = rows_ref[e, pl.ds(16, 16)]
            rows_ref[e, pl.ds(16, 16)] = v1 * exj


def _make_edge_kernel(layer):
    is_l1 = layer == 1
    xs_rows = 2 * N_PAD if is_l1 else N_PAD
    if is_l1:
        out_type = jax.ShapeDtypeStruct((2 * N_PAD, EMB), jnp.float32)
    else:
        out_type = (
            jax.ShapeDtypeStruct((2 * N_PAD, EMB), jnp.float32),
            jax.ShapeDtypeStruct((2 * N_PAD // 4, 8), jnp.float32),
        )

    scratch = [
        pltpu.VMEM_SHARED((N_PAD, EMB), jnp.float32),
        pltpu.VMEM_SHARED((N_PAD // 4, 8), jnp.float32),
        pltpu.VMEM((128,), jnp.int32),
        pltpu.VMEM((128,), jnp.int32),
        pltpu.VMEM((128,), jnp.int32),
        pltpu.VMEM((128,), jnp.int32),
        pltpu.VMEM((128, 8), jnp.float32),
        pltpu.VMEM((128, 8), jnp.float32),
        pltpu.VMEM((128, EMB), jnp.float32),
        pltpu.VMEM((128, 8), jnp.float32),
        pltpu.VMEM((128, 8), jnp.float32),
        pltpu.VMEM((128, EMB), jnp.float32),
        pltpu.VMEM((32, 8), jnp.float32),
        pltpu.VMEM((128,), jnp.int32),
        pltpu.VMEM((64,), jnp.float32),
        pltpu.SemaphoreType.DMA,
    ]

    def body(src2d, dst2d, asdT, xsT, bias64, *rest):
        if is_l1:
            (h_out,) = rest[:1]
            scr = rest[1:]
        else:
            h_out, den_out = rest[:2]
            scr = rest[2:]
        (acc, den, sbuf, dbuf, sidx, didx, asd_s, asd_d, rows, edd, iasd,
         irows, iden, dadx, bbuf, sem) = scr

        c = lax.axis_index("c")
        s = lax.axis_index("s")

        pltpu.sync_copy(bias64, bbuf)

        z16 = jnp.zeros((16,), jnp.float32)
        for col in range(4, 8):
            colv = jnp.full((16,), col, jnp.int32)
            for g in range(8):
                plsc.store_scatter(edd, [_IOTA16() + 16 * g, colv], z16)
            plsc.store_scatter(iden, [_IOTA16(), colv], z16)
            plsc.store_scatter(iden, [_IOTA16() + 16, colv], z16)

        tab_off = c * N_PAD if is_l1 else 0 * c
        iscale = 1.0 if is_l1 else jnp.where(c == 0, 1.0, 0.0).astype(jnp.float32)

        row0_base = s * NPT

        def iden_store(g, idx16, ex):
            plsc.store_scatter(iden, [idx16 // 4, idx16 % 4], ex)

        def init_chunk(k, _):
            row0 = row0_base + k * 128
            pltpu.sync_copy(asdT.at[pl.ds(tab_off + row0, 128)], iasd)
            pltpu.sync_copy(xsT.at[pl.ds(tab_off + row0, 128)], irows)
            _scale_chunk(iasd, _splat(0), iasd, _splat(1), irows, iden_store,
                         iscale)
            pltpu.sync_copy(irows, acc.at[pl.ds(row0, 128)])
            pltpu.sync_copy(iden, den.at[pl.ds(row0 // 4, 32)])
            return 0

        lax.fori_loop(0, NPT // 128, init_chunk, 0, unroll=False)
        plsc.subcore_barrier()

        if is_l1:
            nblk_c = NBLK_E
            blk0_c = 0 * c
        else:
            nblk_c = NBLK_E // 2
            blk0_c = c * (NBLK_E // 2)
        per = nblk_c // 16
        rem = nblk_c - per * 16
        nblk = per + jnp.where(s < rem, 1, 0)
        blk0 = blk0_c + s * per + jnp.minimum(s, rem)

        def edd_store(g, idx16, ex):
            dv = plsc.load_gather(dbuf, [idx16])
            z = jnp.zeros((16,), jnp.float32)
            for col in range(4):
                plsc.store_scatter(edd, [idx16, jnp.full((16,), col, jnp.int32)], z)
            plsc.store_scatter(edd, [idx16, dv % 4], ex)

        def edge_chunk(bi, _):
            blk = blk0 + bi
            pltpu.sync_copy(src2d.at[blk], sbuf)
            pltpu.sync_copy(dst2d.at[blk], dbuf)
            for g in range(8):
                idx16 = _IOTA16() + (16 * g)
                sv = plsc.load_gather(sbuf, [idx16])
                plsc.store_scatter(sidx, [idx16], sv + tab_off)
                dv = plsc.load_gather(dbuf, [idx16])
                plsc.store_scatter(dadx, [idx16], dv + tab_off)
                plsc.store_scatter(didx, [idx16], dv // 4)
            ca = pltpu.async_copy(asdT.at[sidx], asd_s, sem)
            cb = pltpu.async_copy(asdT.at[dadx], asd_d, sem)
            cc = pltpu.async_copy(xsT.at[sidx], rows, sem)
            ca.wait()
            cb.wait()
            cc.wait()
            _scale_chunk(asd_s, _splat(0), asd_d, _splat(1), rows, edd_store, 1.0)
            pltpu.sync_copy(rows, acc.at[dbuf], add=True)
            pltpu.sync_copy(edd, den.at[didx], add=True)
            return 0

        lax.fori_loop(0, nblk, edge_chunk, 0, unroll=False)
        plsc.subcore_barrier()

        def dump_chunk(k, _):
            row0 = row0_base + k * 128
            pltpu.sync_copy(acc.at[pl.ds(row0, 128)], irows)
            pltpu.sync_copy(den.at[pl.ds(row0 // 4, 32)], iden)
            if is_l1:
                b0 = bbuf[pl.ds(32 * c, 16)]
                b1 = bbuf[pl.ds(32 * c + 16, 16)]
                for g in range(8):
                    idx16 = _IOTA16() + (16 * g)
                    dv = plsc.load_gather(iden, [idx16 // 4, idx16 % 4])
                    for j in range(16):
                        e = 16 * g + j
                        dj = dv[j]
                        v0 = irows[e, pl.ds(0, 16)] / dj + b0
                        v1 = irows[e, pl.ds(16, 16)] / dj + b1
                        irows[e, pl.ds(0, 16)] = jnp.where(
                            v0 > 0, v0, jnp.exp(v0) - 1.0)
                        irows[e, pl.ds(16, 16)] = jnp.where(
                            v1 > 0, v1, jnp.exp(v1) - 1.0)
                pltpu.sync_copy(irows, h_out.at[pl.ds(c * N_PAD + row0, 128)])
            else:
                pltpu.sync_copy(irows, h_out.at[pl.ds(c * N_PAD + row0, 128)])
                pltpu.sync_copy(
                    iden, den_out.at[pl.ds((c * N_PAD + row0) // 4, 32)])
            return 0

        lax.fori_loop(0, NPT // 128, dump_chunk, 0, unroll=False)

    return functools.partial(
        pl.kernel,
        out_type=out_type,
        mesh=_SC_MESH,
        compiler_params=_SC_PARAMS,
        scratch_types=scratch,
    )(body)


def _readout_body(uid2d, iid2d, utab, num2, den2, bias32, comb, *scr):
    (ubuf, nbuf, kbuf, mbuf, urows, n0, n1, d0, d1, cbuf, bbuf, sem) = scr
    c = lax.axis_index("c")
    s = lax.axis_index("s")
    wid = s * 2 + c
    pltpu.sync_copy(bias32, bbuf)
    b0 = bbuf[pl.ds(0, 16)]
    b1 = bbuf[pl.ds(16, 16)]

    def do_chunk(q, base_ids, is_item):
        pltpu.sync_copy((iid2d if is_item else uid2d).at[q], ubuf)
        off = 25000 if is_item else 0
        for g in range(8):
            idx16 = _IOTA16() + (16 * g)
            iv = plsc.load_gather(ubuf, [idx16]) + off
            plsc.store_scatter(ubuf, [idx16], iv)
            plsc.store_scatter(nbuf, [idx16], iv + N_PAD)
            plsc.store_scatter(kbuf, [idx16], iv // 4)
            plsc.store_scatter(mbuf, [idx16], iv // 4 + N_PAD // 4)
        ca = pltpu.async_copy(num2.at[ubuf], n0, sem)
        cb = pltpu.async_copy(num2.at[nbuf], n1, sem)
        cc = pltpu.async_copy(den2.at[kbuf], d0, sem)
        cd = pltpu.async_copy(den2.at[mbuf], d1, sem)
        ca.wait(); cb.wait(); cc.wait(); cd.wait()
        col0 = 32 if is_item else 64
        for g in range(8):
            idx16 = _IOTA16() + (16 * g)
            iv = plsc.load_gather(ubuf, [idx16])
            s0 = plsc.load_gather(d0, [idx16, iv % 4])
            s1 = plsc.load_gather(d1, [idx16, iv % 4])
            dsum = s0 + s1
            for j in range(16):
                e = 16 * g + j
                dj = dsum[j]
                v0 = (n0[e, pl.ds(0, 16)] + n1[e, pl.ds(0, 16)]) / dj + b0
                v1 = (n0[e, pl.ds(16, 16)] + n1[e, pl.ds(16, 16)]) / dj + b1
                cbuf[e, pl.ds(col0, 16)] = v0
                cbuf[e, pl.ds(col0 + 16, 16)] = v1

    def chunk_body(k, _):
        q = wid * 4 + k
        pltpu.sync_copy(uid2d.at[q], ubuf)
        pltpu.async_copy(utab.at[ubuf], urows, sem).wait()
        for e_g in range(8):
            for j in range(16):
                e = 16 * e_g + j
                cbuf[e, pl.ds(0, 16)] = urows[e, pl.ds(0, 16)]
                cbuf[e, pl.ds(16, 16)] = urows[e, pl.ds(16, 16)]
        do_chunk(q, None, True)
        do_chunk(q, None, False)
        pltpu.sync_copy(cbuf, comb.at[pl.ds(q * 128, 128)])
        return 0

    lax.fori_loop(0, 4, chunk_body, 0, unroll=False)


def _readout(uid2d, iid2d, utab, num2, den2, bias32):
    scratch = [
        pltpu.VMEM((128,), jnp.int32),
        pltpu.VMEM((128,), jnp.int32),
        pltpu.VMEM((128,), jnp.int32),
        pltpu.VMEM((128,), jnp.int32),
        pltpu.VMEM((128, EMB), jnp.float32),
        pltpu.VMEM((128, EMB), jnp.float32),
        pltpu.VMEM((128, EMB), jnp.float32),
        pltpu.VMEM((128, 8), jnp.float32),
        pltpu.VMEM((128, 8), jnp.float32),
        pltpu.VMEM((128, 3 * EMB), jnp.float32),
        pltpu.VMEM((32,), jnp.float32),
        pltpu.SemaphoreType.DMA,
    ]
    return functools.partial(
        pl.kernel,
        out_type=jax.ShapeDtypeStruct((B, 3 * EMB), jnp.float32),
        mesh=_SC_MESH,
        compiler_params=_SC_PARAMS,
        scratch_types=scratch,
    )(_readout_body)(uid2d, iid2d, utab, num2, den2, bias32)


_edge_l1 = _make_edge_kernel(1)
_edge_l2 = _make_edge_kernel(2)


def kernel(user_ids, item_ids, content_features, edge_index, user_table,
           item_table, Wc, bc, W1s, W1d, a1s, a1d, bconv1, W2s, W2d, a2s,
           a2d, bconv2, Wp1, bp1, Wp2, bp2):
    user_ids = user_ids.astype(jnp.int32)
    item_ids = item_ids.astype(jnp.int32)
    ei = edge_index.astype(jnp.int32)
    src2d = ei[0].reshape(NBLK_E, 128)
    dst2d = ei[1].reshape(NBLK_E, 128)

    c_emb = _cemb(content_features, Wc, bc)

    u_emb = user_table[user_ids]
    i_emb = item_table[item_ids]
    x = jnp.zeros((N_PAD, 2 * EMB), jnp.float32)
    x = x.at[user_ids].set(
        jnp.concatenate([u_emb, jnp.zeros_like(c_emb)], axis=1))
    x = x.at[NUM_USERS + item_ids].set(
        jnp.concatenate([i_emb, c_emb], axis=1))

    xsT1, asdT1 = _l1pre(x, W1s, W1d, a1s[0], a1d[0])
    h1A = _edge_l1(src2d, dst2d, asdT1, xsT1, bconv1)
    xsT2, asdT2 = _l2pre(h1A, W2s, W2d, a2s[0, 0], a2d[0, 0])
    num2, den2 = _edge_l2(src2d, dst2d, asdT2, xsT2,
                          jnp.zeros((64,), jnp.float32))
    uid2d = user_ids.reshape(128, 128)
    iid2d = item_ids.reshape(128, 128)
    comb = _readout(uid2d, iid2d, user_table, num2, den2, bconv2)
    wp2p = jnp.concatenate([Wp2, jnp.zeros((7, EMB), jnp.float32)], axis=0)
    out = _mlp(comb, Wp1, bp1, wp2p, bp2)
    return out[:, 0]
